# Initial kernel scaffold; baseline (speedup 1.0000x reference)
import jax
import jax.numpy as jnp
from jax.experimental import pallas as pl


def kernel(x, edge_index, W1, b1, W2, b2):
    raise NotImplementedError("write your pallas kernel here")


# speedup vs baseline: 16.8245x; 16.8245x over previous
import functools

import jax
import jax.numpy as jnp
from jax import lax
from jax.experimental import pallas as pl
from jax.experimental.pallas import tpu as pltpu
from jax.experimental.pallas import tpu_sc as plsc

_NC = 2
_NS = 16
_NW = _NC * _NS

_CH = 80
_NBUF = 5


def _segment_sum(vals, src3, dst3, n, d):
    nch = src3.shape[1]
    npad = 10240
    rows_sub = npad // _NS
    zrows = 128
    nzc = rows_sub // zrows
    mesh = plsc.VectorSubcoreMesh(
        core_axis_name="c", subcore_axis_name="s",
        num_cores=_NC, num_subcores=_NS)

    @functools.partial(
        pl.kernel,
        out_type=jax.ShapeDtypeStruct((_NC, npad, d), jnp.float32),
        mesh=mesh,
        scratch_types=[
            pltpu.VMEM((nch, _CH), jnp.int32),
            pltpu.VMEM((nch, _CH), jnp.int32),
            pltpu.VMEM((_NBUF, _CH, d), jnp.float32),
            pltpu.VMEM((zrows, d), jnp.float32),
            pltpu.VMEM_SHARED((npad, d), jnp.float32),
        ] + [pltpu.SemaphoreType.DMA] * _NBUF,
        compiler_params=pltpu.CompilerParams(use_tc_tiling_on_sc=False),
    )
    def agg(vals_hbm, src_hbm, dst_hbm, out_hbm,
            src_v, dst_v, rows_v, zero_v, acc_sh, *sems):
        cid = lax.axis_index("c")
        sid = lax.axis_index("s")
        wid = cid * _NS + sid

        def _zrow(i, carry):
            for k in range(d // 16):
                zero_v[i, pl.ds(16 * k, 16)] = jnp.zeros((16,), jnp.float32)
            return carry
        lax.fori_loop(0, zrows, _zrow, 0)

        def _zcopy(k, carry):
            pltpu.sync_copy(
                zero_v, acc_sh.at[pl.ds(sid * rows_sub + k * zrows, zrows)])
            return carry
        lax.fori_loop(0, nzc, _zcopy, 0)

        pltpu.sync_copy(src_hbm.at[wid], src_v)
        pltpu.sync_copy(dst_hbm.at[wid], dst_v)

        plsc.subcore_barrier()

        def _gather(j, b):
            return pltpu.make_async_copy(
                vals_hbm.at[src_v.at[j]], rows_v.at[b], sems[b])

        for b in range(_NBUF - 1):
            _gather(b, b).start()

        def _group(g, carry):
            for b in range(_NBUF):
                j = g * _NBUF + b
                _gather(j, b).wait()
                nxt = j + _NBUF - 1

                @pl.when(nxt < nch)
                def _():
                    _gather(nxt, (b + _NBUF - 1) % _NBUF).start()

                pltpu.sync_copy(rows_v.at[b], acc_sh.at[dst_v.at[j]], add=True)
            return carry
        lax.fori_loop(0, nch // _NBUF, _group, 0)

        plsc.subcore_barrier()
        pltpu.sync_copy(
            acc_sh.at[pl.ds(sid * rows_sub, rows_sub)],
            out_hbm.at[cid, pl.ds(sid * rows_sub, rows_sub)])

    return agg(vals, src3, dst3)[:, :n, :]


def _matmul(x, w):
    def body(x_ref, w_ref, o_ref):
        o_ref[...] = jnp.dot(x_ref[...], w_ref[...],
                             preferred_element_type=jnp.float32)
    return pl.pallas_call(
        body,
        out_shape=jax.ShapeDtypeStruct((x.shape[0], w.shape[1]), jnp.float32),
    )(x, w)


def _fused_relu_matmul(p, a0, a1, b1r, w2):
    def body(p_ref, a0_ref, a1_ref, b_ref, w_ref, o_ref):
        h = jnp.maximum(
            p_ref[...] + a0_ref[...] + a1_ref[...] + b_ref[...], 0.0)
        o_ref[...] = jnp.dot(h, w_ref[...], preferred_element_type=jnp.float32)
    return pl.pallas_call(
        body,
        out_shape=jax.ShapeDtypeStruct((p.shape[0], w2.shape[1]), jnp.float32),
    )(p, a0, a1, b1r, w2)


def _final_add(q, a0, a1, b2r):
    def body(q_ref, a0_ref, a1_ref, b_ref, o_ref):
        o_ref[...] = q_ref[...] + a0_ref[...] + a1_ref[...] + b_ref[...]
    return pl.pallas_call(
        body,
        out_shape=jax.ShapeDtypeStruct(q.shape, jnp.float32),
    )(q, a0, a1, b2r)


def kernel(x, edge_index, W1, b1, W2, b2):
    n = x.shape[0]
    e = edge_index.shape[1]
    h = W1.shape[1]
    c = W2.shape[1]
    epw = e // _NW
    nch = epw // _CH

    src3 = edge_index[0].reshape(_NW, nch, _CH)
    dst3 = edge_index[1].reshape(_NW, nch, _CH)

    p = _matmul(x, W1)
    a1 = _segment_sum(p, src3, dst3, n, h)
    q = _fused_relu_matmul(p, a1[0], a1[1], b1.reshape(1, h), W2)
    a2 = _segment_sum(q, src3, dst3, n, c)
    return _final_add(q, a2[0], a2[1], b2.reshape(1, c))


# speedup vs baseline: 18.2500x; 1.0847x over previous
import functools

import jax
import jax.numpy as jnp
from jax import lax
from jax.experimental import pallas as pl
from jax.experimental.pallas import tpu as pltpu
from jax.experimental.pallas import tpu_sc as plsc

_NC = 2
_NS = 16
_NW = _NC * _NS

_CH = 80
_NBUF = 5


def _segment_sum(vals, src3, dst3, n, d):
    nch = src3.shape[1]
    npad = 10240
    rows_sub = npad // _NS
    zrows = 128
    nzc = rows_sub // zrows
    mesh = plsc.VectorSubcoreMesh(
        core_axis_name="c", subcore_axis_name="s",
        num_cores=_NC, num_subcores=_NS)

    @functools.partial(
        pl.kernel,
        out_type=jax.ShapeDtypeStruct((_NC, npad, d), jnp.float32),
        mesh=mesh,
        scratch_types=[
            pltpu.VMEM((nch, _CH), jnp.int32),
            pltpu.VMEM((nch, _CH), jnp.int32),
            pltpu.VMEM((_NBUF, _CH, d), jnp.float32),
            pltpu.VMEM((zrows, d), jnp.float32),
            pltpu.VMEM_SHARED((npad, d), jnp.float32),
        ] + [pltpu.SemaphoreType.DMA] * (2 * _NBUF),
        compiler_params=pltpu.CompilerParams(use_tc_tiling_on_sc=False),
    )
    def agg(vals_hbm, src_hbm, dst_hbm, out_hbm,
            src_v, dst_v, rows_v, zero_v, acc_sh, *sems):
        cid = lax.axis_index("c")
        sid = lax.axis_index("s")
        wid = cid * _NS + sid

        def _zrow(i, carry):
            for k in range(d // 16):
                zero_v[i, pl.ds(16 * k, 16)] = jnp.zeros((16,), jnp.float32)
            return carry
        lax.fori_loop(0, zrows, _zrow, 0)

        def _zcopy(k, carry):
            pltpu.sync_copy(
                zero_v, acc_sh.at[pl.ds(sid * rows_sub + k * zrows, zrows)])
            return carry
        lax.fori_loop(0, nzc, _zcopy, 0)

        pltpu.sync_copy(src_hbm.at[wid], src_v)
        pltpu.sync_copy(dst_hbm.at[wid], dst_v)

        plsc.subcore_barrier()

        def _gather(j, b):
            return pltpu.make_async_copy(
                vals_hbm.at[src_v.at[j]], rows_v.at[b], sems[b])

        def _scatter_start(j, b):
            pltpu.async_copy(
                rows_v.at[b], acc_sh.at[dst_v.at[j]], sems[_NBUF + b],
                add=True)

        def _scatter_wait(j, b):
            pltpu.make_async_copy(
                rows_v.at[b], acc_sh.at[dst_v.at[j]], sems[_NBUF + b]).wait()

        for b in range(_NBUF - 1):
            _gather(b, b).start()

        def _group(g, carry):
            for b in range(_NBUF):
                j = g * _NBUF + b
                _gather(j, b).wait()
                nxt = j + _NBUF - 1
                bn = (b + _NBUF - 1) % _NBUF

                @pl.when(j >= 1)
                def _():
                    _scatter_wait(j - 1, bn)

                @pl.when(nxt < nch)
                def _():
                    _gather(nxt, bn).start()

                _scatter_start(j, b)
            return carry
        lax.fori_loop(0, nch // _NBUF, _group, 0)

        _scatter_wait(nch - 1, (nch - 1) % _NBUF)
        plsc.subcore_barrier()
        pltpu.sync_copy(
            acc_sh.at[pl.ds(sid * rows_sub, rows_sub)],
            out_hbm.at[cid, pl.ds(sid * rows_sub, rows_sub)])

    return agg(vals, src3, dst3)


def _matmul(x, w):
    def body(x_ref, w_ref, o_ref):
        o_ref[...] = jnp.dot(x_ref[...], w_ref[...],
                             preferred_element_type=jnp.float32)
    return pl.pallas_call(
        body,
        out_shape=jax.ShapeDtypeStruct((x.shape[0], w.shape[1]), jnp.float32),
    )(x, w)


def _fused_relu_matmul(p, a, b1r, w2):
    n = p.shape[0]

    def body(p_ref, a_ref, b_ref, w_ref, o_ref):
        agg = a_ref[0, :n, :] + a_ref[1, :n, :]
        h = jnp.maximum(p_ref[...] + agg + b_ref[...], 0.0)
        o_ref[...] = jnp.dot(h, w_ref[...], preferred_element_type=jnp.float32)
    return pl.pallas_call(
        body,
        out_shape=jax.ShapeDtypeStruct((n, w2.shape[1]), jnp.float32),
    )(p, a, b1r, w2)


def _final_add(q, a, b2r):
    n = q.shape[0]

    def body(q_ref, a_ref, b_ref, o_ref):
        o_ref[...] = q_ref[...] + a_ref[0, :n, :] + a_ref[1, :n, :] + b_ref[...]
    return pl.pallas_call(
        body,
        out_shape=jax.ShapeDtypeStruct(q.shape, jnp.float32),
    )(q, a, b2r)


def kernel(x, edge_index, W1, b1, W2, b2):
    n = x.shape[0]
    e = edge_index.shape[1]
    h = W1.shape[1]
    c = W2.shape[1]
    epw = e // _NW
    nch = epw // _CH

    src3 = edge_index[0].reshape(_NW, nch, _CH)
    dst3 = edge_index[1].reshape(_NW, nch, _CH)

    p = _matmul(x, W1)
    a1 = _segment_sum(p, src3, dst3, n, h)
    q = _fused_relu_matmul(p, a1, b1.reshape(1, h), W2)
    a2 = _segment_sum(q, src3, dst3, n, c)
    return _final_add(q, a2, b2.reshape(1, c))


# speedup vs baseline: 21.9908x; 1.2050x over previous
import functools

import jax
import jax.numpy as jnp
from jax import lax
from jax.experimental import pallas as pl
from jax.experimental.pallas import tpu as pltpu
from jax.experimental.pallas import tpu_sc as plsc

_NC = 2
_NS = 16
_NW = _NC * _NS

_N = 10000
_NPAD = 10240
_E = 320000
_EPAD = 327680
_CH = 128
_NBUF = 5
_ROWS_PER_W = _EPAD // _EPAD * 0 + (_EPAD // _CH) // _NW


def _segment_sum(vals, src2, dst2, d):
    nch = _ROWS_PER_W
    rows_sub = _NPAD // _NS
    zrows = 128
    nzc = rows_sub // zrows
    mesh = plsc.VectorSubcoreMesh(
        core_axis_name="c", subcore_axis_name="s",
        num_cores=_NC, num_subcores=_NS)

    @functools.partial(
        pl.kernel,
        out_type=jax.ShapeDtypeStruct((_NC, _NPAD, d), jnp.float32),
        mesh=mesh,
        scratch_types=[
            pltpu.VMEM((nch, _CH), jnp.int32),
            pltpu.VMEM((nch, _CH), jnp.int32),
            pltpu.VMEM((_NBUF, _CH, d), jnp.float32),
            pltpu.VMEM((zrows, d), jnp.float32),
            pltpu.VMEM_SHARED((_NPAD, d), jnp.float32),
        ] + [pltpu.SemaphoreType.DMA] * (2 * _NBUF),
        compiler_params=pltpu.CompilerParams(use_tc_tiling_on_sc=False),
    )
    def agg(vals_hbm, src_hbm, dst_hbm, out_hbm,
            src_v, dst_v, rows_v, zero_v, acc_sh, *sems):
        cid = lax.axis_index("c")
        sid = lax.axis_index("s")
        wid = cid * _NS + sid

        def _zrow(i, carry):
            for k in range(d // 16):
                zero_v[i, pl.ds(16 * k, 16)] = jnp.zeros((16,), jnp.float32)
            return carry
        lax.fori_loop(0, zrows, _zrow, 0)

        def _zcopy(k, carry):
            pltpu.sync_copy(
                zero_v, acc_sh.at[pl.ds(sid * rows_sub + k * zrows, zrows)])
            return carry
        lax.fori_loop(0, nzc, _zcopy, 0)

        pltpu.sync_copy(src_hbm.at[pl.ds(wid * nch, nch)], src_v)
        pltpu.sync_copy(dst_hbm.at[pl.ds(wid * nch, nch)], dst_v)

        plsc.subcore_barrier()

        def _gather(j, b):
            return pltpu.make_async_copy(
                vals_hbm.at[src_v.at[j]], rows_v.at[b], sems[b])

        def _scatter_start(j, b):
            pltpu.async_copy(
                rows_v.at[b], acc_sh.at[dst_v.at[j]], sems[_NBUF + b],
                add=True)

        def _scatter_wait(j, b):
            pltpu.make_async_copy(
                rows_v.at[b], acc_sh.at[dst_v.at[j]], sems[_NBUF + b]).wait()

        for b in range(_NBUF - 1):
            _gather(b, b).start()

        def _group(g, carry):
            for b in range(_NBUF):
                j = g * _NBUF + b
                _gather(j, b).wait()
                nxt = j + _NBUF - 1
                bn = (b + _NBUF - 1) % _NBUF

                @pl.when(j >= 1)
                def _():
                    _scatter_wait(j - 1, bn)

                @pl.when(nxt < nch)
                def _():
                    _gather(nxt, bn).start()

                _scatter_start(j, b)
            return carry
        lax.fori_loop(0, nch // _NBUF, _group, 0)

        _scatter_wait(nch - 1, (nch - 1) % _NBUF)
        plsc.subcore_barrier()
        pltpu.sync_copy(
            acc_sh.at[pl.ds(sid * rows_sub, rows_sub)],
            out_hbm.at[cid, pl.ds(sid * rows_sub, rows_sub)])

    return agg(vals, src2, dst2)


def _matmul(x2, w1b):
    def body(x_ref, w_ref, o_ref):
        o_ref[...] = jnp.dot(x_ref[...], w_ref[...],
                             preferred_element_type=jnp.float32)
    return pl.pallas_call(
        body,
        out_shape=jax.ShapeDtypeStruct((x2.shape[0], w1b.shape[1]),
                                       jnp.float32),
    )(x2, w1b)


def _fused_relu_matmul(p2, a1t, b1b, w2b):
    def body(p_ref, a_ref, b_ref, w_ref, o_ref):
        h2 = jnp.maximum(
            p_ref[...] + a_ref[0, :5000, :] + a_ref[1, :5000, :] + b_ref[...],
            0.0)
        q2 = jnp.dot(h2, w_ref[...], preferred_element_type=jnp.float32)
        o_ref[...] = jnp.concatenate(
            [q2, jnp.zeros((120, 32), jnp.float32)], axis=0)
    return pl.pallas_call(
        body,
        out_shape=jax.ShapeDtypeStruct((5120, 32), jnp.float32),
    )(p2, a1t, b1b, w2b)


def _final_add(q8, a2t, b2t):
    def body(q_ref, a_ref, b_ref, o_ref):
        o_ref[...] = (q_ref[:1250, :] + a_ref[0, :1250, :]
                      + a_ref[1, :1250, :] + b_ref[...])
    return pl.pallas_call(
        body,
        out_shape=jax.ShapeDtypeStruct((1250, 128), jnp.float32),
    )(q8, a2t, b2t)


def kernel(x, edge_index, W1, b1, W2, b2):
    f = x.shape[1]
    h = W1.shape[1]
    c = W2.shape[1]
    npadextra = _EPAD - _E

    pad_iota = jnp.arange(npadextra, dtype=jnp.int32)
    src2 = jnp.concatenate([edge_index[0], pad_iota % 256]).reshape(-1, _CH)
    dst2 = jnp.concatenate(
        [edge_index[1], _N + pad_iota % (_NPAD - _N)]).reshape(-1, _CH)

    w1b = jnp.zeros((2 * f, 2 * h), jnp.float32)
    w1b = w1b.at[:f, :h].set(W1).at[f:, h:].set(W1)
    w2b = jnp.zeros((2 * h, 2 * c), jnp.float32)
    w2b = w2b.at[:h, :c].set(W2).at[h:, c:].set(W2)
    b1b = jnp.concatenate([b1, b1]).reshape(1, 2 * h)
    b2t = jnp.tile(b2, 8).reshape(1, 128)

    x2 = x.reshape(_N // 2, 2 * f)
    p2 = _matmul(x2, w1b)
    a1 = _segment_sum(p2.reshape(_N, h), src2, dst2, h)
    a1t = a1.reshape(_NC, _NPAD * h // 128, 128)
    qp = _fused_relu_matmul(p2, a1t, b1b, w2b)
    qlin = qp.reshape(_NPAD, c)
    a2 = _segment_sum(qlin, src2, dst2, c)
    a2t = a2.reshape(_NC, _NPAD * c // 128, 128)
    q8 = qlin.reshape(_NPAD * c // 128, 128)
    out8 = _final_add(q8, a2t, b2t)
    return out8.reshape(_N, c)


# speedup vs baseline: 23.3350x; 1.0611x over previous
import functools

import jax
import jax.numpy as jnp
from jax import lax
from jax.experimental import pallas as pl
from jax.experimental.pallas import tpu as pltpu
from jax.experimental.pallas import tpu_sc as plsc

_NC = 2
_NS = 16
_NW = _NC * _NS

_N = 10000
_NPAD = 10240
_E = 320000
_EPAD = 327680
_CH = 128
_NBUF = 5
_ROWS_PER_W = _EPAD // _EPAD * 0 + (_EPAD // _CH) // _NW


def _segment_sum(vals, e3, d):
    nch = _ROWS_PER_W
    rows_sub = _NPAD // _NS
    zrows = 128
    nzc = rows_sub // zrows
    mesh = plsc.VectorSubcoreMesh(
        core_axis_name="c", subcore_axis_name="s",
        num_cores=_NC, num_subcores=_NS)

    @functools.partial(
        pl.kernel,
        out_type=jax.ShapeDtypeStruct((_NC, _NPAD, d), jnp.float32),
        mesh=mesh,
        scratch_types=[
            pltpu.VMEM((nch, _CH), jnp.int32),
            pltpu.VMEM((nch, _CH), jnp.int32),
            pltpu.VMEM((_NBUF, _CH, d), jnp.float32),
            pltpu.VMEM((zrows, d), jnp.float32),
            pltpu.VMEM_SHARED((_NPAD, d), jnp.float32),
        ] + [pltpu.SemaphoreType.DMA] * (2 * _NBUF),
        compiler_params=pltpu.CompilerParams(use_tc_tiling_on_sc=False),
    )
    def agg(vals_hbm, e_hbm, out_hbm,
            src_v, dst_v, rows_v, zero_v, acc_sh, *sems):
        cid = lax.axis_index("c")
        sid = lax.axis_index("s")
        wid = cid * _NS + sid

        def _zrow(i, carry):
            for k in range(d // 16):
                zero_v[i, pl.ds(16 * k, 16)] = jnp.zeros((16,), jnp.float32)
            return carry
        lax.fori_loop(0, zrows, _zrow, 0)

        def _zcopy(k, carry):
            pltpu.sync_copy(
                zero_v, acc_sh.at[pl.ds(sid * rows_sub + k * zrows, zrows)])
            return carry
        lax.fori_loop(0, nzc, _zcopy, 0)

        pltpu.sync_copy(e_hbm.at[0, pl.ds(wid * nch, nch)], src_v)
        pltpu.sync_copy(e_hbm.at[1, pl.ds(wid * nch, nch)], dst_v)

        plsc.subcore_barrier()

        def _gather(j, b):
            return pltpu.make_async_copy(
                vals_hbm.at[src_v.at[j]], rows_v.at[b], sems[b])

        def _scatter_start(j, b):
            pltpu.async_copy(
                rows_v.at[b], acc_sh.at[dst_v.at[j]], sems[_NBUF + b],
                add=True)

        def _scatter_wait(j, b):
            pltpu.make_async_copy(
                rows_v.at[b], acc_sh.at[dst_v.at[j]], sems[_NBUF + b]).wait()

        for b in range(_NBUF - 1):
            _gather(b, b).start()

        def _group(g, carry):
            for b in range(_NBUF):
                j = g * _NBUF + b
                _gather(j, b).wait()
                nxt = j + _NBUF - 1
                bn = (b + _NBUF - 1) % _NBUF

                @pl.when(j >= 1)
                def _():
                    _scatter_wait(j - 1, bn)

                @pl.when(nxt < nch)
                def _():
                    _gather(nxt, bn).start()

                _scatter_start(j, b)
            return carry
        lax.fori_loop(0, nch // _NBUF, _group, 0)

        _scatter_wait(nch - 1, (nch - 1) % _NBUF)
        plsc.subcore_barrier()
        pltpu.sync_copy(
            acc_sh.at[pl.ds(sid * rows_sub, rows_sub)],
            out_hbm.at[cid, pl.ds(sid * rows_sub, rows_sub)])

    return agg(vals, e3)


def _matmul(x2, w1):
    def body(x_ref, w_ref, o_ref):
        w = w_ref[...]
        pa = jnp.dot(x_ref[:, :128], w, preferred_element_type=jnp.float32)
        pb = jnp.dot(x_ref[:, 128:], w, preferred_element_type=jnp.float32)
        o_ref[...] = jnp.concatenate([pa, pb], axis=1)
    return pl.pallas_call(
        body,
        out_shape=jax.ShapeDtypeStruct((x2.shape[0], 128), jnp.float32),
    )(x2, w1)


def _fused_relu_matmul(p2, a1t, b1b, w2):
    def body(p_ref, a_ref, b_ref, w_ref, o_ref):
        h2 = jnp.maximum(
            p_ref[...] + a_ref[0, :5000, :] + a_ref[1, :5000, :] + b_ref[...],
            0.0)
        w = w_ref[...]
        qa = jnp.dot(h2[:, :64], w, preferred_element_type=jnp.float32)
        qb = jnp.dot(h2[:, 64:], w, preferred_element_type=jnp.float32)
        q2 = jnp.concatenate([qa, qb], axis=1)
        o_ref[...] = jnp.concatenate(
            [q2, jnp.zeros((120, 32), jnp.float32)], axis=0)
    return pl.pallas_call(
        body,
        out_shape=jax.ShapeDtypeStruct((5120, 32), jnp.float32),
    )(p2, a1t, b1b, w2)


def _final_add(q8, a2t, b2t):
    def body(q_ref, a_ref, b_ref, o_ref):
        o_ref[...] = (q_ref[:1250, :] + a_ref[0, :1250, :]
                      + a_ref[1, :1250, :] + b_ref[...])
    return pl.pallas_call(
        body,
        out_shape=jax.ShapeDtypeStruct((1250, 128), jnp.float32),
    )(q8, a2t, b2t)


def kernel(x, edge_index, W1, b1, W2, b2):
    f = x.shape[1]
    h = W1.shape[1]
    c = W2.shape[1]
    npadextra = _EPAD - _E

    pad_iota = jnp.arange(npadextra, dtype=jnp.int32)
    epad = jnp.stack([pad_iota % 256, _N + pad_iota % (_NPAD - _N)])
    e3 = jnp.concatenate([edge_index, epad], axis=1).reshape(2, -1, _CH)

    b1b = jnp.concatenate([b1, b1]).reshape(1, 2 * h)
    b2t = jnp.tile(b2, 8).reshape(1, 128)

    x2 = x.reshape(_N // 2, 2 * f)
    p2 = _matmul(x2, W1)
    a1 = _segment_sum(p2.reshape(_N, h), e3, h)
    a1t = a1.reshape(_NC, _NPAD * h // 128, 128)
    qp = _fused_relu_matmul(p2, a1t, b1b, W2)
    qlin = qp.reshape(_NPAD, c)
    a2 = _segment_sum(qlin, e3, c)
    a2t = a2.reshape(_NC, _NPAD * c // 128, 128)
    q8 = qlin.reshape(_NPAD * c // 128, 128)
    out8 = _final_add(q8, a2t, b2t)
    return out8.reshape(_N, c)


# speedup vs baseline: 23.5845x; 1.0107x over previous
import functools

import jax
import jax.numpy as jnp
from jax import lax
from jax.experimental import pallas as pl
from jax.experimental.pallas import tpu as pltpu
from jax.experimental.pallas import tpu_sc as plsc

_NC = 2
_NS = 16
_NW = _NC * _NS

_N = 10000
_NPAD = 10240
_E = 320000
_EPAD = 327680
_CH = 128
_NBUF = 5
_ROWS_PER_W = _EPAD // _EPAD * 0 + (_EPAD // _CH) // _NW


def _segment_sum(vals, e3, d):
    nch = _ROWS_PER_W
    rows_sub = _NPAD // _NS
    zrows = 128
    nzc = rows_sub // zrows
    n = vals.shape[0]
    stage = (n * d) <= 200_000
    mesh = plsc.VectorSubcoreMesh(
        core_axis_name="c", subcore_axis_name="s",
        num_cores=_NC, num_subcores=_NS)

    @functools.partial(
        pl.kernel,
        out_type=jax.ShapeDtypeStruct((_NC, _NPAD, d), jnp.float32),
        mesh=mesh,
        scratch_types=[
            pltpu.VMEM((nch, _CH), jnp.int32),
            pltpu.VMEM((nch, _CH), jnp.int32),
            pltpu.VMEM((_NBUF, _CH, d), jnp.float32),
            pltpu.VMEM((zrows, d), jnp.float32),
            pltpu.VMEM_SHARED((_NPAD, d), jnp.float32),
        ] + ([pltpu.VMEM_SHARED((n, d), jnp.float32)] if stage else [])
          + [pltpu.SemaphoreType.DMA] * (2 * _NBUF),
        compiler_params=pltpu.CompilerParams(use_tc_tiling_on_sc=False),
    )
    def agg(vals_hbm, e_hbm, out_hbm,
            src_v, dst_v, rows_v, zero_v, acc_sh, *rest):
        vals_sh = rest[0] if stage else None
        sems = rest[1:] if stage else rest
        cid = lax.axis_index("c")
        sid = lax.axis_index("s")
        wid = cid * _NS + sid

        def _zrow(i, carry):
            for k in range(d // 16):
                zero_v[i, pl.ds(16 * k, 16)] = jnp.zeros((16,), jnp.float32)
            return carry
        lax.fori_loop(0, zrows, _zrow, 0)

        def _zcopy(k, carry):
            pltpu.sync_copy(
                zero_v, acc_sh.at[pl.ds(sid * rows_sub + k * zrows, zrows)])
            return carry
        lax.fori_loop(0, nzc, _zcopy, 0)

        pltpu.sync_copy(e_hbm.at[0, pl.ds(wid * nch, nch)], src_v)
        pltpu.sync_copy(e_hbm.at[1, pl.ds(wid * nch, nch)], dst_v)

        if stage:
            pltpu.sync_copy(vals_hbm.at[pl.ds(sid * (n // _NS), n // _NS)],
                            vals_sh.at[pl.ds(sid * (n // _NS), n // _NS)])

        plsc.subcore_barrier()

        gsrc = vals_sh if stage else vals_hbm

        def _gather(j, b):
            return pltpu.make_async_copy(
                gsrc.at[src_v.at[j]], rows_v.at[b], sems[b])

        def _scatter_start(j, b):
            pltpu.async_copy(
                rows_v.at[b], acc_sh.at[dst_v.at[j]], sems[_NBUF + b],
                add=True)

        def _scatter_wait(j, b):
            pltpu.make_async_copy(
                rows_v.at[b], acc_sh.at[dst_v.at[j]], sems[_NBUF + b]).wait()

        for b in range(_NBUF - 1):
            _gather(b, b).start()

        def _group(g, carry):
            for b in range(_NBUF):
                j = g * _NBUF + b
                _gather(j, b).wait()
                nxt = j + _NBUF - 1
                bn = (b + _NBUF - 1) % _NBUF

                @pl.when(j >= 1)
                def _():
                    _scatter_wait(j - 1, bn)

                @pl.when(nxt < nch)
                def _():
                    _gather(nxt, bn).start()

                _scatter_start(j, b)
            return carry
        lax.fori_loop(0, nch // _NBUF, _group, 0)

        _scatter_wait(nch - 1, (nch - 1) % _NBUF)
        plsc.subcore_barrier()
        pltpu.sync_copy(
            acc_sh.at[pl.ds(sid * rows_sub, rows_sub)],
            out_hbm.at[cid, pl.ds(sid * rows_sub, rows_sub)])

    return agg(vals, e3)


def _matmul(x2, w1):
    def body(x_ref, w_ref, o_ref):
        w = w_ref[...]
        pa = jnp.dot(x_ref[:, :128], w, preferred_element_type=jnp.float32)
        pb = jnp.dot(x_ref[:, 128:], w, preferred_element_type=jnp.float32)
        o_ref[...] = jnp.concatenate([pa, pb], axis=1)
    return pl.pallas_call(
        body,
        out_shape=jax.ShapeDtypeStruct((x2.shape[0], 128), jnp.float32),
    )(x2, w1)


def _fused_relu_matmul(p2, a1t, b1b, w2):
    def body(p_ref, a_ref, b_ref, w_ref, o_ref):
        h2 = jnp.maximum(
            p_ref[...] + a_ref[0, :5000, :] + a_ref[1, :5000, :] + b_ref[...],
            0.0)
        w = w_ref[...]
        qa = jnp.dot(h2[:, :64], w, preferred_element_type=jnp.float32)
        qb = jnp.dot(h2[:, 64:], w, preferred_element_type=jnp.float32)
        q2 = jnp.concatenate([qa, qb], axis=1)
        o_ref[...] = jnp.concatenate(
            [q2, jnp.zeros((120, 32), jnp.float32)], axis=0)
    return pl.pallas_call(
        body,
        out_shape=jax.ShapeDtypeStruct((5120, 32), jnp.float32),
    )(p2, a1t, b1b, w2)


def _final_add(q8, a2t, b2t):
    def body(q_ref, a_ref, b_ref, o_ref):
        o_ref[...] = (q_ref[:1250, :] + a_ref[0, :1250, :]
                      + a_ref[1, :1250, :] + b_ref[...])
    return pl.pallas_call(
        body,
        out_shape=jax.ShapeDtypeStruct((1250, 128), jnp.float32),
    )(q8, a2t, b2t)


def kernel(x, edge_index, W1, b1, W2, b2):
    f = x.shape[1]
    h = W1.shape[1]
    c = W2.shape[1]
    npadextra = _EPAD - _E

    pad_iota = jnp.arange(npadextra, dtype=jnp.int32)
    epad = jnp.stack([pad_iota % 256, _N + pad_iota % (_NPAD - _N)])
    e3 = jnp.concatenate([edge_index, epad], axis=1).reshape(2, -1, _CH)

    b1b = jnp.concatenate([b1, b1]).reshape(1, 2 * h)
    b2t = jnp.tile(b2, 8).reshape(1, 128)

    x2 = x.reshape(_N // 2, 2 * f)
    p2 = _matmul(x2, W1)
    a1 = _segment_sum(p2.reshape(_N, h), e3, h)
    a1t = a1.reshape(_NC, _NPAD * h // 128, 128)
    qp = _fused_relu_matmul(p2, a1t, b1b, W2)
    qlin = qp.reshape(_NPAD, c)
    a2 = _segment_sum(qlin, e3, c)
    a2t = a2.reshape(_NC, _NPAD * c // 128, 128)
    q8 = qlin.reshape(_NPAD * c // 128, 128)
    out8 = _final_add(q8, a2t, b2t)
    return out8.reshape(_N, c)


# speedup vs baseline: 24.2946x; 1.0301x over previous
import functools

import jax
import jax.numpy as jnp
from jax import lax
from jax.experimental import pallas as pl
from jax.experimental.pallas import tpu as pltpu
from jax.experimental.pallas import tpu_sc as plsc

_NC = 2
_NS = 16
_NW = _NC * _NS

_N = 10000
_NPAD = 10240
_E = 320000
_EPAD = 327680
_CH = 128
_NBUF = 5
_ROWS_PER_W = _EPAD // _EPAD * 0 + (_EPAD // _CH) // _NW


def _segment_sum(vals, e3, d):
    nch = _ROWS_PER_W
    rows_sub = _NPAD // _NS
    zrows = 128
    nzc = rows_sub // zrows
    n = vals.shape[0]
    stage = (n * d) <= 200_000
    mesh = plsc.VectorSubcoreMesh(
        core_axis_name="c", subcore_axis_name="s",
        num_cores=_NC, num_subcores=_NS)

    @functools.partial(
        pl.kernel,
        out_type=jax.ShapeDtypeStruct((_NC, _NPAD, d), jnp.float32),
        mesh=mesh,
        scratch_types=[
            pltpu.VMEM((nch, _CH), jnp.int32),
            pltpu.VMEM((nch, _CH), jnp.int32),
            pltpu.VMEM((_NBUF, _CH, d), jnp.float32),
            pltpu.VMEM((zrows, d), jnp.float32),
            pltpu.VMEM_SHARED((_NPAD, d), jnp.float32),
        ] + ([pltpu.VMEM_SHARED((n, d), jnp.float32)] if stage else [])
          + [pltpu.SemaphoreType.DMA] * (2 * _NBUF),
        compiler_params=pltpu.CompilerParams(use_tc_tiling_on_sc=False),
    )
    def agg(vals_hbm, e_hbm, out_hbm,
            src_v, dst_v, rows_v, zero_v, acc_sh, *rest):
        vals_sh = rest[0] if stage else None
        sems = rest[1:] if stage else rest
        cid = lax.axis_index("c")
        sid = lax.axis_index("s")
        wid = cid * _NS + sid

        def _zrow(i, carry):
            for k in range(d // 16):
                zero_v[i, pl.ds(16 * k, 16)] = jnp.zeros((16,), jnp.float32)
            return carry
        lax.fori_loop(0, zrows, _zrow, 0)

        def _zcopy(k, carry):
            pltpu.sync_copy(
                zero_v, acc_sh.at[pl.ds(sid * rows_sub + k * zrows, zrows)])
            return carry
        lax.fori_loop(0, nzc, _zcopy, 0)

        pltpu.sync_copy(e_hbm.at[0, pl.ds(wid * nch, nch)], src_v)
        pltpu.sync_copy(e_hbm.at[1, pl.ds(wid * nch, nch)], dst_v)

        if stage:
            pltpu.sync_copy(vals_hbm.at[pl.ds(sid * (n // _NS), n // _NS)],
                            vals_sh.at[pl.ds(sid * (n // _NS), n // _NS)])

        plsc.subcore_barrier()

        gsrc = vals_sh if stage else vals_hbm

        def _gather(j, b):
            return pltpu.make_async_copy(
                gsrc.at[src_v.at[j]], rows_v.at[b], sems[b])

        def _scatter_start(j, b):
            pltpu.async_copy(
                rows_v.at[b], acc_sh.at[dst_v.at[j]], sems[_NBUF + b],
                add=True)

        def _scatter_wait(j, b):
            pltpu.make_async_copy(
                rows_v.at[b], acc_sh.at[dst_v.at[j]], sems[_NBUF + b]).wait()

        for b in range(_NBUF - 1):
            _gather(b, b).start()

        def _group(g, carry):
            for b in range(_NBUF):
                j = g * _NBUF + b
                _gather(j, b).wait()
                nxt = j + _NBUF - 1
                bn = (b + _NBUF - 1) % _NBUF

                @pl.when(j >= 1)
                def _():
                    _scatter_wait(j - 1, bn)

                @pl.when(nxt < nch)
                def _():
                    _gather(nxt, bn).start()

                _scatter_start(j, b)
            return carry
        lax.fori_loop(0, nch // _NBUF, _group, 0)

        _scatter_wait(nch - 1, (nch - 1) % _NBUF)
        plsc.subcore_barrier()
        pltpu.sync_copy(
            acc_sh.at[pl.ds(sid * rows_sub, rows_sub)],
            out_hbm.at[cid, pl.ds(sid * rows_sub, rows_sub)])

    return agg(vals, e3)


def _matmul(x, w1):
    def body(x_ref, w_ref, o_ref):
        x3 = x_ref[...].reshape(5000, 2, 128)
        w = w_ref[...]
        pa = jnp.dot(x3[:, 0, :], w, preferred_element_type=jnp.float32)
        pb = jnp.dot(x3[:, 1, :], w, preferred_element_type=jnp.float32)
        o_ref[...] = jnp.concatenate([pa, pb], axis=1)
    return pl.pallas_call(
        body,
        out_shape=jax.ShapeDtypeStruct((x.shape[0] // 2, 128), jnp.float32),
    )(x, w1)


def _fused_relu_matmul(p2, a1t, b1b, w2):
    def body(p_ref, a_ref, b_ref, w_ref, o_ref):
        h2 = jnp.maximum(
            p_ref[...] + a_ref[0, :5000, :] + a_ref[1, :5000, :] + b_ref[...],
            0.0)
        w = w_ref[...]
        qa = jnp.dot(h2[:, :64], w, preferred_element_type=jnp.float32)
        qb = jnp.dot(h2[:, 64:], w, preferred_element_type=jnp.float32)
        q2 = jnp.concatenate([qa, qb], axis=1)
        o_ref[...] = jnp.concatenate(
            [q2, jnp.zeros((120, 32), jnp.float32)], axis=0)
    return pl.pallas_call(
        body,
        out_shape=jax.ShapeDtypeStruct((5120, 32), jnp.float32),
    )(p2, a1t, b1b, w2)


def _final_add(q8, a2t, b2t):
    def body(q_ref, a_ref, b_ref, o_ref):
        o_ref[...] = (q_ref[:1250, :] + a_ref[0, :1250, :]
                      + a_ref[1, :1250, :] + b_ref[...])
    return pl.pallas_call(
        body,
        out_shape=jax.ShapeDtypeStruct((1250, 128), jnp.float32),
    )(q8, a2t, b2t)


def kernel(x, edge_index, W1, b1, W2, b2):
    f = x.shape[1]
    h = W1.shape[1]
    c = W2.shape[1]
    npadextra = _EPAD - _E

    pad_iota = jnp.arange(npadextra, dtype=jnp.int32)
    epad = jnp.stack([pad_iota % 256, _N + pad_iota % (_NPAD - _N)])
    e3 = jnp.concatenate([edge_index, epad], axis=1).reshape(2, -1, _CH)

    b1b = jnp.concatenate([b1, b1]).reshape(1, 2 * h)
    b2t = jnp.tile(b2, 8).reshape(1, 128)

    p2 = _matmul(x, W1)
    a1 = _segment_sum(p2.reshape(_N, h), e3, h)
    a1t = a1.reshape(_NC, _NPAD * h // 128, 128)
    qp = _fused_relu_matmul(p2, a1t, b1b, W2)
    qlin = qp.reshape(_NPAD, c)
    a2 = _segment_sum(qlin, e3, c)
    a2t = a2.reshape(_NC, _NPAD * c // 128, 128)
    q8 = qlin.reshape(_NPAD * c // 128, 128)
    out8 = _final_add(q8, a2t, b2t)
    return out8.reshape(_N, c)


# speedup vs baseline: 25.3564x; 1.0437x over previous
import functools

import jax
import jax.numpy as jnp
from jax import lax
from jax.experimental import pallas as pl
from jax.experimental.pallas import tpu as pltpu
from jax.experimental.pallas import tpu_sc as plsc

_NC = 2
_NS = 16
_NW = _NC * _NS

_N = 10000
_NPAD = 10240
_E = 320000
_EPAD = 327680
_CH = 128
_NBUF = 5
_ROWS_PER_W = _EPAD // _EPAD * 0 + (_EPAD // _CH) // _NW


def _segment_sum(vals, e3, d):
    nch = _ROWS_PER_W
    rows_sub = _NPAD // _NS
    zrows = 128
    nzc = rows_sub // zrows
    n = vals.shape[0]
    dt = vals.dtype
    lanes = 32 if dt == jnp.bfloat16 else 16
    stage = (n * d) <= 200_000
    mesh = plsc.VectorSubcoreMesh(
        core_axis_name="c", subcore_axis_name="s",
        num_cores=_NC, num_subcores=_NS)

    @functools.partial(
        pl.kernel,
        out_type=jax.ShapeDtypeStruct((_NC, _NPAD, d), dt),
        mesh=mesh,
        scratch_types=[
            pltpu.VMEM((nch, _CH), jnp.int32),
            pltpu.VMEM((nch, _CH), jnp.int32),
            pltpu.VMEM((_NBUF, _CH, d), dt),
            pltpu.VMEM((zrows, d), dt),
            pltpu.VMEM_SHARED((_NPAD, d), dt),
        ] + ([pltpu.VMEM_SHARED((n, d), dt)] if stage else [])
          + [pltpu.SemaphoreType.DMA] * (2 * _NBUF),
        compiler_params=pltpu.CompilerParams(use_tc_tiling_on_sc=False),
    )
    def agg(vals_hbm, e_hbm, out_hbm,
            src_v, dst_v, rows_v, zero_v, acc_sh, *rest):
        vals_sh = rest[0] if stage else None
        sems = rest[1:] if stage else rest
        cid = lax.axis_index("c")
        sid = lax.axis_index("s")
        wid = cid * _NS + sid

        def _zrow(i, carry):
            for k in range(d // lanes):
                zero_v[i, pl.ds(lanes * k, lanes)] = jnp.zeros((lanes,), dt)
            return carry
        lax.fori_loop(0, zrows, _zrow, 0)

        def _zcopy(k, carry):
            pltpu.sync_copy(
                zero_v, acc_sh.at[pl.ds(sid * rows_sub + k * zrows, zrows)])
            return carry
        lax.fori_loop(0, nzc, _zcopy, 0)

        pltpu.sync_copy(e_hbm.at[0, pl.ds(wid * nch, nch)], src_v)
        pltpu.sync_copy(e_hbm.at[1, pl.ds(wid * nch, nch)], dst_v)

        if stage:
            pltpu.sync_copy(vals_hbm.at[pl.ds(sid * (n // _NS), n // _NS)],
                            vals_sh.at[pl.ds(sid * (n // _NS), n // _NS)])

        plsc.subcore_barrier()

        gsrc = vals_sh if stage else vals_hbm

        def _gather(j, b):
            return pltpu.make_async_copy(
                gsrc.at[src_v.at[j]], rows_v.at[b], sems[b])

        def _scatter_start(j, b):
            pltpu.async_copy(
                rows_v.at[b], acc_sh.at[dst_v.at[j]], sems[_NBUF + b],
                add=True)

        def _scatter_wait(j, b):
            pltpu.make_async_copy(
                rows_v.at[b], acc_sh.at[dst_v.at[j]], sems[_NBUF + b]).wait()

        for b in range(_NBUF - 1):
            _gather(b, b).start()

        def _group(g, carry):
            for b in range(_NBUF):
                j = g * _NBUF + b
                _gather(j, b).wait()
                nxt = j + _NBUF - 1
                bn = (b + _NBUF - 1) % _NBUF

                @pl.when(j >= 1)
                def _():
                    _scatter_wait(j - 1, bn)

                @pl.when(nxt < nch)
                def _():
                    _gather(nxt, bn).start()

                _scatter_start(j, b)
            return carry
        lax.fori_loop(0, nch // _NBUF, _group, 0)

        _scatter_wait(nch - 1, (nch - 1) % _NBUF)
        plsc.subcore_barrier()
        pltpu.sync_copy(
            acc_sh.at[pl.ds(sid * rows_sub, rows_sub)],
            out_hbm.at[cid, pl.ds(sid * rows_sub, rows_sub)])

    return agg(vals, e3)


def _matmul(x, w1):
    def body(x_ref, w_ref, o_ref):
        x3 = x_ref[...].reshape(5000, 2, 128)
        w = w_ref[...]
        pa = jnp.dot(x3[:, 0, :], w, preferred_element_type=jnp.float32)
        pb = jnp.dot(x3[:, 1, :], w, preferred_element_type=jnp.float32)
        p2 = jnp.concatenate([pa, pb], axis=1).astype(jnp.bfloat16)
        o_ref[...] = jnp.concatenate(
            [p2, jnp.zeros((120, 128), jnp.bfloat16)], axis=0)
    return pl.pallas_call(
        body,
        out_shape=jax.ShapeDtypeStruct((5120, 128), jnp.bfloat16),
    )(x, w1)


def _fused_relu_matmul(p2, a1t, b1b, w2):
    def body(p_ref, a_ref, b_ref, w_ref, o_ref):
        h2 = jnp.maximum(
            p_ref[:5000, :].astype(jnp.float32)
            + a_ref[0, :5000, :].astype(jnp.float32)
            + a_ref[1, :5000, :].astype(jnp.float32) + b_ref[...],
            0.0)
        w = w_ref[...]
        qa = jnp.dot(h2[:, :64], w, preferred_element_type=jnp.float32)
        qb = jnp.dot(h2[:, 64:], w, preferred_element_type=jnp.float32)
        q2 = jnp.concatenate([qa, qb], axis=1)
        o_ref[...] = jnp.concatenate(
            [q2, jnp.zeros((120, 32), jnp.float32)], axis=0)
    return pl.pallas_call(
        body,
        out_shape=jax.ShapeDtypeStruct((5120, 32), jnp.float32),
    )(p2, a1t, b1b, w2)


def _final_add(q8, a2t, b2t):
    def body(q_ref, a_ref, b_ref, o_ref):
        o_ref[...] = (q_ref[:1250, :] + a_ref[0, :1250, :]
                      + a_ref[1, :1250, :] + b_ref[...])
    return pl.pallas_call(
        body,
        out_shape=jax.ShapeDtypeStruct((1250, 128), jnp.float32),
    )(q8, a2t, b2t)


def kernel(x, edge_index, W1, b1, W2, b2):
    f = x.shape[1]
    h = W1.shape[1]
    c = W2.shape[1]
    npadextra = _EPAD - _E

    pad_iota = jnp.arange(npadextra, dtype=jnp.int32)
    epad = jnp.stack([pad_iota % 256, _N + pad_iota % (_NPAD - _N)])
    e3 = jnp.concatenate([edge_index, epad], axis=1).reshape(2, -1, _CH)

    b1b = jnp.concatenate([b1, b1]).reshape(1, 2 * h)
    b2t = jnp.tile(b2, 8).reshape(1, 128)

    p2 = _matmul(x, W1)
    a1 = _segment_sum(p2.reshape(_NPAD, h), e3, h)
    a1t = a1.reshape(_NC, _NPAD * h // 128, 128)
    qp = _fused_relu_matmul(p2, a1t, b1b, W2)
    qlin = qp.reshape(_NPAD, c)
    a2 = _segment_sum(qlin, e3, c)
    a2t = a2.reshape(_NC, _NPAD * c // 128, 128)
    q8 = qlin.reshape(_NPAD * c // 128, 128)
    out8 = _final_add(q8, a2t, b2t)
    return out8.reshape(_N, c)


# speedup vs baseline: 25.3641x; 1.0003x over previous
import functools

import jax
import jax.numpy as jnp
from jax import lax
from jax.experimental import pallas as pl
from jax.experimental.pallas import tpu as pltpu
from jax.experimental.pallas import tpu_sc as plsc

_NC = 2
_NS = 16
_NW = _NC * _NS

_N = 10000
_NPAD = 10240
_E = 320000
_EPAD = 327680
_CH = 128
_NBUF = 5
_ROWS_PER_W = (_EPAD // _CH) // _NW


def _segment_sum(vals, e3, d):
    nch = _ROWS_PER_W
    rows_sub = _NPAD // _NS
    zrows = 128
    nzc = rows_sub // zrows
    n = vals.shape[0]
    dt = vals.dtype
    lanes = 32 if dt == jnp.bfloat16 else 16
    stage = (n * d) <= 200_000
    mesh = plsc.VectorSubcoreMesh(
        core_axis_name="c", subcore_axis_name="s",
        num_cores=_NC, num_subcores=_NS)

    @functools.partial(
        pl.kernel,
        out_type=jax.ShapeDtypeStruct((_NC, _NPAD, d), dt),
        mesh=mesh,
        scratch_types=[
            pltpu.VMEM((nch, _CH), jnp.int32),
            pltpu.VMEM((nch, _CH), jnp.int32),
            pltpu.VMEM((_NBUF, _CH, d), dt),
            pltpu.VMEM((zrows, d), dt),
            pltpu.VMEM_SHARED((_NPAD, d), dt),
        ] + ([pltpu.VMEM_SHARED((n, d), dt)] if stage else [])
          + [pltpu.SemaphoreType.DMA] * (2 * _NBUF),
        compiler_params=pltpu.CompilerParams(use_tc_tiling_on_sc=False),
    )
    def agg(vals_hbm, e_hbm, out_hbm,
            src_v, dst_v, rows_v, zero_v, acc_sh, *rest):
        vals_sh = rest[0] if stage else None
        sems = rest[1:] if stage else rest
        cid = lax.axis_index("c")
        sid = lax.axis_index("s")
        wid = cid * _NS + sid

        def _zrow(i, carry):
            for k in range(d // lanes):
                zero_v[i, pl.ds(lanes * k, lanes)] = jnp.zeros((lanes,), dt)
            return carry
        lax.fori_loop(0, zrows, _zrow, 0)

        def _zcopy(k, carry):
            pltpu.sync_copy(
                zero_v, acc_sh.at[pl.ds(sid * rows_sub + k * zrows, zrows)])
            return carry
        lax.fori_loop(0, nzc, _zcopy, 0)

        pltpu.sync_copy(e_hbm.at[0, pl.ds(wid * nch, nch)], src_v)
        pltpu.sync_copy(e_hbm.at[1, pl.ds(wid * nch, nch)], dst_v)

        if stage:
            pltpu.sync_copy(vals_hbm.at[pl.ds(sid * (n // _NS), n // _NS)],
                            vals_sh.at[pl.ds(sid * (n // _NS), n // _NS)])

        plsc.subcore_barrier()

        gsrc = vals_sh if stage else vals_hbm

        def _gather(j, b):
            return pltpu.make_async_copy(
                gsrc.at[src_v.at[j]], rows_v.at[b], sems[b])

        def _scatter_start(j, b):
            pltpu.async_copy(
                rows_v.at[b], acc_sh.at[dst_v.at[j]], sems[_NBUF + b],
                add=True)

        def _scatter_wait(j, b):
            pltpu.make_async_copy(
                rows_v.at[b], acc_sh.at[dst_v.at[j]], sems[_NBUF + b]).wait()

        for b in range(_NBUF - 1):
            _gather(b, b).start()

        def _group(g, carry):
            for b in range(_NBUF):
                j = g * _NBUF + b
                _gather(j, b).wait()
                nxt = j + _NBUF - 1
                bn = (b + _NBUF - 1) % _NBUF

                @pl.when(j >= 1)
                def _():
                    _scatter_wait(j - 1, bn)

                @pl.when(nxt < nch)
                def _():
                    _gather(nxt, bn).start()

                _scatter_start(j, b)
            return carry
        lax.fori_loop(0, nch // _NBUF, _group, 0)

        _scatter_wait(nch - 1, (nch - 1) % _NBUF)
        plsc.subcore_barrier()
        pltpu.sync_copy(
            acc_sh.at[pl.ds(sid * rows_sub, rows_sub)],
            out_hbm.at[cid, pl.ds(sid * rows_sub, rows_sub)])

    return agg(vals, e3)


def _matmul(x, w1):
    def body(x_ref, w_ref, o_ref):
        x3 = x_ref[...].reshape(5000, 2, 128)
        w = w_ref[...]
        pa = jnp.dot(x3[:, 0, :], w, preferred_element_type=jnp.float32)
        pb = jnp.dot(x3[:, 1, :], w, preferred_element_type=jnp.float32)
        p2 = jnp.concatenate([pa, pb], axis=1).astype(jnp.bfloat16)
        o_ref[...] = jnp.concatenate(
            [p2, jnp.zeros((120, 128), jnp.bfloat16)], axis=0)
    return pl.pallas_call(
        body,
        out_shape=jax.ShapeDtypeStruct((5120, 128), jnp.bfloat16),
    )(x, w1)


def _fused_relu_matmul(p2, a1t, b1b, w2):
    def body(p_ref, a_ref, b_ref, w_ref, o_ref):
        h2 = jnp.maximum(
            p_ref[:5000, :].astype(jnp.float32)
            + a_ref[0, :5000, :].astype(jnp.float32)
            + a_ref[1, :5000, :].astype(jnp.float32) + b_ref[...],
            0.0)
        w = w_ref[...]
        qa = jnp.dot(h2[:, :64], w, preferred_element_type=jnp.float32)
        qb = jnp.dot(h2[:, 64:], w, preferred_element_type=jnp.float32)
        q2 = jnp.concatenate([qa, qb], axis=1)
        o_ref[...] = jnp.concatenate(
            [q2, jnp.zeros((120, 32), jnp.float32)], axis=0)
    return pl.pallas_call(
        body,
        out_shape=jax.ShapeDtypeStruct((5120, 32), jnp.float32),
    )(p2, a1t, b1b, w2)


def _final_add(q8, a2t, b2t):
    def body(q_ref, a_ref, b_ref, o_ref):
        o_ref[...] = (q_ref[:1250, :] + a_ref[0, :1250, :]
                      + a_ref[1, :1250, :] + b_ref[...])
    return pl.pallas_call(
        body,
        out_shape=jax.ShapeDtypeStruct((1250, 128), jnp.float32),
    )(q8, a2t, b2t)


def kernel(x, edge_index, W1, b1, W2, b2):
    h = W1.shape[1]
    c = W2.shape[1]
    npadextra = _EPAD - _E

    pad_iota = jnp.arange(npadextra, dtype=jnp.int32)
    epad = jnp.stack([pad_iota % 256, _N + pad_iota % (_NPAD - _N)])
    e3 = jnp.concatenate([edge_index, epad], axis=1).reshape(2, -1, _CH)

    b1b = jnp.concatenate([b1, b1]).reshape(1, 2 * h)
    b2t = jnp.tile(b2, 8).reshape(1, 128)

    p2 = _matmul(x, W1)
    a1 = _segment_sum(p2.reshape(_NPAD, h), e3, h)
    a1t = a1.reshape(_NC, _NPAD * h // 128, 128)
    qp = _fused_relu_matmul(p2, a1t, b1b, W2)
    qlin = qp.reshape(_NPAD, c)
    a2 = _segment_sum(qlin, e3, c)
    a2t = a2.reshape(_NC, _NPAD * c // 128, 128)
    q8 = qlin.reshape(_NPAD * c // 128, 128)
    out8 = _final_add(q8, a2t, b2t)
    return out8.reshape(_N, c)
